# baseline (speedup 1.0000x reference)
import jax
import jax.numpy as jnp
from jax.experimental import pallas as pl
from jax.experimental.pallas import tpu as pltpu
from jax.experimental.pallas import tpu_sc as plsc

VOCAB = 1000000
D_MODEL = 64
SCALE = 8.0
IDX_ROW = 128
GATHERS = 4
WINDOW = IDX_ROW * GATHERS


LUT_LANES = 6400
PACK_ROWS = 64 * ((VOCAB + 127) // 128)


def _pack_scale_lut(lut_t):
    grid = (VOCAB + LUT_LANES - 1) // LUT_LANES

    def body(x_ref, o_ref):
        t = (x_ref[...] * SCALE).T
        for k in range(LUT_LANES // 128):
            o_ref[64 * k : 64 * (k + 1)] = jnp.concatenate(
                [t[128 * k : 128 * k + 64], t[128 * k + 64 : 128 * (k + 1)]],
                axis=1,
            )

    return pl.pallas_call(
        body,
        grid=(grid,),
        in_specs=[pl.BlockSpec((D_MODEL, LUT_LANES), lambda i: (0, i))],
        out_specs=pl.BlockSpec((LUT_LANES // 2, 128), lambda i: (i, 0)),
        out_shape=jax.ShapeDtypeStruct((PACK_ROWS, 128), jnp.float32),
    )(lut_t)


IDX_BLK = 128


def _remap_idx(idx2d):
    rows = idx2d.shape[0]

    def body(x_ref, o_ref):
        v = x_ref[...]
        o_ref[...] = 128 * (v >> 7) + 2 * (v & 63) + ((v >> 6) & 1)

    return pl.pallas_call(
        body,
        grid=(rows // IDX_BLK,),
        in_specs=[pl.BlockSpec((IDX_BLK, IDX_ROW), lambda i: (i, 0))],
        out_specs=pl.BlockSpec((IDX_BLK, IDX_ROW), lambda i: (i, 0)),
        out_shape=jax.ShapeDtypeStruct(idx2d.shape, jnp.int32),
    )(idx2d)


def _sc_gather(lut_lin, idx2d, n):
    vector_mesh = plsc.VectorSubcoreMesh(
        core_axis_name="core", subcore_axis_name="subcore"
    )

    @pl.kernel(
        out_type=jax.ShapeDtypeStruct((n, D_MODEL), jnp.float32),
        mesh=vector_mesh,
        scratch_types=[pltpu.SemaphoreType.DMA],
        compiler_params=pltpu.CompilerParams(use_tc_tiling_on_sc=False),
    )
    def run(lut_hbm, i_hbm, o_hbm, sem):
        def body(i_vmem, o_vmem):
            copies = [
                pltpu.async_copy(
                    lut_hbm.at[i_vmem.at[j]],
                    o_vmem.at[pl.ds(j * IDX_ROW, IDX_ROW)],
                    sem,
                )
                for j in range(GATHERS)
            ]
            for c in copies:
                c.wait()

        pltpu.emit_pipeline(
            body,
            grid=(n // WINDOW,),
            in_specs=[pl.BlockSpec((GATHERS, IDX_ROW), lambda i: (i, 0))],
            out_specs=[pl.BlockSpec((WINDOW, D_MODEL), lambda i: (i, 0))],
            core_axis_name=("core", "subcore"),
            dimension_semantics=(pltpu.PARALLEL,),
        )(i_hbm, o_hbm)

    return run(lut_lin, idx2d)


B_BLOCK = 128
S_BLOCK = 8


def _out_transpose(flat3d):
    b, s, _ = flat3d.shape

    def body(x_ref, o_ref):
        for k in range(S_BLOCK):
            o_ref[k] = x_ref[:, k, :].T

    return pl.pallas_call(
        body,
        grid=(b // B_BLOCK, s // S_BLOCK),
        in_specs=[
            pl.BlockSpec((B_BLOCK, S_BLOCK, D_MODEL), lambda i, j: (i, j, 0))
        ],
        out_specs=pl.BlockSpec(
            (S_BLOCK, D_MODEL, B_BLOCK), lambda i, j: (j, 0, i)
        ),
        out_shape=jax.ShapeDtypeStruct((s, D_MODEL, b), jnp.float32),
    )(flat3d)


def kernel(x, lut):
    b, s = x.shape
    n = b * s
    packed = _pack_scale_lut(lut.T)
    lut_lin = packed.reshape(2 * PACK_ROWS, D_MODEL)
    idx2d = _remap_idx(x.reshape(n // IDX_ROW, IDX_ROW).astype(jnp.int32))
    flat = _sc_gather(lut_lin, idx2d, n)
    t2 = _out_transpose(flat.reshape(b, s, D_MODEL))
    return jnp.transpose(t2, (2, 0, 1))


# speedup vs baseline: 1.0485x; 1.0485x over previous
import jax
import jax.numpy as jnp
from jax.experimental import pallas as pl
from jax.experimental.pallas import tpu as pltpu
from jax.experimental.pallas import tpu_sc as plsc

VOCAB = 1000000
D_MODEL = 64
SCALE = 8.0
IDX_ROW = 128
GATHERS = 4
WINDOW = IDX_ROW * GATHERS


LUT_LANES = 6400
PACK_ROWS = 64 * ((VOCAB + 127) // 128)


def _pack_scale_lut(lut_t):
    grid = (VOCAB + LUT_LANES - 1) // LUT_LANES

    def body(x_ref, o_ref):
        t = (x_ref[...] * SCALE).T
        for k in range(LUT_LANES // 128):
            o_ref[64 * k : 64 * (k + 1)] = jnp.concatenate(
                [t[128 * k : 128 * k + 64], t[128 * k + 64 : 128 * (k + 1)]],
                axis=1,
            )

    return pl.pallas_call(
        body,
        grid=(grid,),
        in_specs=[pl.BlockSpec((D_MODEL, LUT_LANES), lambda i: (0, i))],
        out_specs=pl.BlockSpec((LUT_LANES // 2, 128), lambda i: (i, 0)),
        out_shape=jax.ShapeDtypeStruct((PACK_ROWS, 128), jnp.float32),
    )(lut_t)


def _prep_idx(xt):
    s, b = xt.shape

    def body(x_ref, o_ref):
        v = x_ref[...]
        p = 128 * (v >> 7) + 2 * (v & 63) + ((v >> 6) & 1)
        l = jax.lax.broadcasted_iota(jnp.int32, (128, 128), 0)
        j = jax.lax.broadcasted_iota(jnp.int32, (128, 128), 1)
        g = (j == 2 * (l & 63) + ((l >> 6) & 1)).astype(jnp.float32)
        permuted = jnp.dot(
            p.astype(jnp.float32),
            g,
            preferred_element_type=jnp.float32,
            precision=jax.lax.Precision.HIGHEST,
        )
        o_ref[...] = permuted.astype(jnp.int32)

    return pl.pallas_call(
        body,
        grid=(b // 128,),
        in_specs=[pl.BlockSpec((s, 128), lambda i: (0, i))],
        out_specs=pl.BlockSpec((s, 128), lambda i: (0, i)),
        out_shape=jax.ShapeDtypeStruct((s, b), jnp.int32),
    )(xt)


def _sc_gather(lut_lin, idx2d, n):
    vector_mesh = plsc.VectorSubcoreMesh(
        core_axis_name="core", subcore_axis_name="subcore"
    )

    @pl.kernel(
        out_type=jax.ShapeDtypeStruct((n, D_MODEL), jnp.float32),
        mesh=vector_mesh,
        scratch_types=[pltpu.SemaphoreType.DMA],
        compiler_params=pltpu.CompilerParams(use_tc_tiling_on_sc=False),
    )
    def run(lut_hbm, i_hbm, o_hbm, sem):
        def body(i_vmem, o_vmem):
            copies = [
                pltpu.async_copy(
                    lut_hbm.at[i_vmem.at[j]],
                    o_vmem.at[pl.ds(j * IDX_ROW, IDX_ROW)],
                    sem,
                )
                for j in range(GATHERS)
            ]
            for c in copies:
                c.wait()

        pltpu.emit_pipeline(
            body,
            grid=(n // WINDOW,),
            in_specs=[pl.BlockSpec((GATHERS, IDX_ROW), lambda i: (i, 0))],
            out_specs=[pl.BlockSpec((WINDOW, D_MODEL), lambda i: (i, 0))],
            core_axis_name=("core", "subcore"),
            dimension_semantics=(pltpu.PARALLEL,),
        )(i_hbm, o_hbm)

    return run(lut_lin, idx2d)


S_BLOCK = 8
Q_BLOCK = 256


def _out_transpose(g3):
    s, q, _ = g3.shape
    b = 2 * q

    def body(x_ref, o_ref):
        for k in range(S_BLOCK):
            for c in range(Q_BLOCK // 64):
                t = x_ref[k, 64 * c : 64 * (c + 1), :]
                o_ref[k, :, 128 * c : 128 * c + 64] = t[:, :D_MODEL].T
                o_ref[k, :, 128 * c + 64 : 128 * (c + 1)] = t[:, D_MODEL:].T

    return pl.pallas_call(
        body,
        grid=(s // S_BLOCK, q // Q_BLOCK),
        in_specs=[
            pl.BlockSpec((S_BLOCK, Q_BLOCK, 128), lambda i, j: (i, j, 0))
        ],
        out_specs=pl.BlockSpec(
            (S_BLOCK, D_MODEL, 2 * Q_BLOCK), lambda i, j: (i, 0, j)
        ),
        out_shape=jax.ShapeDtypeStruct((s, D_MODEL, b), jnp.float32),
    )(g3)


def kernel(x, lut):
    b, s = x.shape
    n = b * s
    packed = _pack_scale_lut(lut.T)
    lut_lin = packed.reshape(2 * PACK_ROWS, D_MODEL)
    idx = _prep_idx(jnp.transpose(x).astype(jnp.int32))
    flat = _sc_gather(lut_lin, idx.reshape(n // IDX_ROW, IDX_ROW), n)
    t2 = _out_transpose(flat.reshape(s, b // 2, 128))
    return jnp.transpose(t2, (2, 0, 1))
